# baseline (speedup 1.0000x reference)
import dataclasses
import functools

import jax
import jax.numpy as jnp
from jax.experimental import pallas as pl
from jax.experimental.pallas import tpu as pltpu
from jax.experimental.pallas import tpu_sc as plsc

N_ATOMS = 10000
N_EDGES = 320000
D = 128
N_RBF = 20

NUM_SC = 2
NUM_SUBCORES = 16
NUM_WORKERS = NUM_SC * NUM_SUBCORES
CHUNK = 80
CHUNKS_PER_WORKER = 125
NC0 = 177
NC1 = 73
E_PAD = NUM_WORKERS * CHUNKS_PER_WORKER * CHUNK
A_PAD = 10112
ROWS_PER_SUB = A_PAD // NUM_SUBCORES

_LN2 = 0.6931471805599453


def _ssp(v):
    return jnp.maximum(v, 0.0) + jnp.log1p(jnp.exp(-jnp.abs(v))) - _LN2


def _h_body(x_ref, w_ref, b_ref, o_ref):
    o_ref[...] = jax.lax.dot_general(
        x_ref[...], w_ref[...], (((1,), (1,)), ((), ())),
        preferred_element_type=jnp.float32) + b_ref[...]


def _compute_h(x, W_in, b_in):
    return pl.pallas_call(
        _h_body,
        out_shape=jax.ShapeDtypeStruct((N_ATOMS, D), jnp.float32),
    )(x, W_in, b_in.reshape(1, D))


_BE = 2000


def _filter_body(fij_ref, rcut_ref, w1_ref, b1_ref, w2_ref, b2_ref, o_ref):
    t = jax.lax.dot_general(
        fij_ref[...].astype(jnp.bfloat16),
        w1_ref[...].astype(jnp.bfloat16), (((1,), (1,)), ((), ())),
        preferred_element_type=jnp.float32) + b1_ref[...]
    t = _ssp(t)
    w = jax.lax.dot_general(
        t.astype(jnp.bfloat16),
        w2_ref[...].astype(jnp.bfloat16), (((1,), (1,)), ((), ())),
        preferred_element_type=jnp.float32) + b2_ref[...]
    o_ref[...] = w * rcut_ref[...]


def _compute_wij(f_ij_p, rcut_p, W_f1, b_f1, W_f2, b_f2):
    grid = (E_PAD // _BE,)
    return pl.pallas_call(
        _filter_body,
        grid=grid,
        in_specs=[
            pl.BlockSpec((_BE, N_RBF), lambda i: (i, 0)),
            pl.BlockSpec((_BE, 1), lambda i: (i, 0)),
            pl.BlockSpec((D, N_RBF), lambda i: (0, 0)),
            pl.BlockSpec((1, D), lambda i: (0, 0)),
            pl.BlockSpec((D, D), lambda i: (0, 0)),
            pl.BlockSpec((1, D), lambda i: (0, 0)),
        ],
        out_specs=pl.BlockSpec((_BE, D), lambda i: (i, 0)),
        out_shape=jax.ShapeDtypeStruct((E_PAD, D), jnp.float32),
    )(f_ij_p, rcut_p, W_f1, b_f1.reshape(1, D), W_f2, b_f2.reshape(1, D))


def _sc_compiler_params():
    cp = pltpu.CompilerParams()
    if "needs_layout_passes" in pltpu.CompilerParams.__dataclass_fields__:
        cp = dataclasses.replace(cp, needs_layout_passes=False)
    return cp


def _sc_aggregate(idx_i2, idx_j2, wij, h):
    mesh = plsc.VectorSubcoreMesh(
        core_axis_name="core", subcore_axis_name="subcore",
        num_cores=NUM_SC, num_subcores=NUM_SUBCORES)

    @functools.partial(
        pl.kernel,
        out_type=jax.ShapeDtypeStruct((NUM_SC, A_PAD, D), jnp.float32),
        mesh=mesh,
        scratch_types=[
            pltpu.VMEM_SHARED((A_PAD, D), jnp.float32),
            [pltpu.VMEM((CHUNK, D), jnp.float32)] * 2,
            [pltpu.VMEM((CHUNK, D), jnp.float32)] * 2,
            [pltpu.VMEM((CHUNK,), jnp.int32)] * 4,
            [pltpu.VMEM((CHUNK,), jnp.int32)] * 4,
            [pltpu.SemaphoreType.DMA] * 2,
            [pltpu.SemaphoreType.DMA] * 2,
            [pltpu.SemaphoreType.DMA] * 2,
            [pltpu.SemaphoreType.DMA] * 4,
        ],
        compiler_params=_sc_compiler_params(),
    )
    def body(idxi_hbm, idxj_hbm, wij_hbm, h_hbm, out_hbm,
             agg_sh, gbufs, wbufs, ib, jb, sg, sw, ss, si):
        core = jax.lax.axis_index("core")
        sub = jax.lax.axis_index("subcore")
        base = jnp.where(core == 0, sub * NC0, NUM_SUBCORES * NC0 + sub * NC1)
        nc = jnp.where(core == 0, NC0, NC1)

        zero16 = jnp.zeros((16,), jnp.float32)
        g0 = gbufs[0]

        @plsc.parallel_loop(0, CHUNK, unroll=2)
        def _(r):
            for l in range(0, D, 16):
                g0[r, pl.ds(l, 16)] = zero16

        row0 = sub * ROWS_PER_SUB

        @pl.loop(0, ROWS_PER_SUB // CHUNK)
        def _(k):
            pltpu.sync_copy(g0, agg_sh.at[pl.ds(row0 + k * CHUNK, CHUNK)])

        _rem = ROWS_PER_SUB % CHUNK
        if _rem:
            pltpu.sync_copy(
                g0.at[pl.ds(0, _rem)],
                agg_sh.at[pl.ds(row0 + (ROWS_PER_SUB // CHUNK) * CHUNK, _rem)])

        plsc.subcore_barrier()

        def issue_idx(c, m4):
            pltpu.async_copy(idxi_hbm.at[base + c], ib[m4], si[m4])
            pltpu.async_copy(idxj_hbm.at[base + c], jb[m4], si[m4])

        def wait_idx(m4):
            pltpu.make_async_copy(idxi_hbm.at[0], ib[m4], si[m4]).wait()
            pltpu.make_async_copy(idxj_hbm.at[0], jb[m4], si[m4]).wait()

        def issue_in(c, m2, m4):
            pltpu.async_copy(h_hbm.at[jb[m4]], gbufs[m2], sg[m2])
            pltpu.async_copy(
                wij_hbm.at[pl.ds((base + c) * CHUNK, CHUNK)], wbufs[m2],
                sw[m2])

        def wait_in(m2):
            pltpu.make_async_copy(
                h_hbm.at[pl.ds(0, CHUNK)], gbufs[m2], sg[m2]).wait()
            pltpu.make_async_copy(
                wij_hbm.at[pl.ds(0, CHUNK)], wbufs[m2], sw[m2]).wait()

        def mul(m2):
            g, w = gbufs[m2], wbufs[m2]

            @plsc.parallel_loop(0, CHUNK, unroll=2)
            def _(r):
                for l in range(0, D, 16):
                    g[r, pl.ds(l, 16)] = g[r, pl.ds(l, 16)] * w[r, pl.ds(l, 16)]

        def issue_scatter(m2, m4):
            pltpu.async_copy(gbufs[m2], agg_sh.at[ib[m4]], ss[m2], add=True)

        def wait_scatter(m2):
            pltpu.make_async_copy(
                h_hbm.at[pl.ds(0, CHUNK)], gbufs[m2], ss[m2]).wait()

        def slot(c, m2, m4, sc_wait=True, idx_wait=True,
                 prefetch_in=True, prefetch_idx=True):
            wait_in(m2)
            if sc_wait:
                wait_scatter(1 - m2)
            if prefetch_in:
                if idx_wait:
                    wait_idx((m4 + 1) % 4)
                issue_in(c + 1, 1 - m2, (m4 + 1) % 4)
            mul(m2)
            issue_scatter(m2, m4)
            if prefetch_idx:
                issue_idx(c + 2, (m4 + 2) % 4)

        pltpu.sync_copy(idxi_hbm.at[base], ib[0])
        pltpu.sync_copy(idxj_hbm.at[base], jb[0])
        pltpu.sync_copy(idxi_hbm.at[base + 1], ib[1])
        pltpu.sync_copy(idxj_hbm.at[base + 1], jb[1])
        issue_in(0, 0, 0)
        slot(0, 0, 0, sc_wait=False, idx_wait=False)
        slot(1, 1, 1)

        @pl.loop(0, (nc - 5) // 4)
        def _(k):
            c = 4 * k + 2
            slot(c, 0, 2)
            slot(c + 1, 1, 3)
            slot(c + 2, 0, 0)
            slot(c + 3, 1, 1)

        slot(nc - 3, 0, 2)
        slot(nc - 2, 1, 3, prefetch_idx=False)
        slot(nc - 1, 0, 0, prefetch_in=False, prefetch_idx=False)
        wait_scatter(0)

        plsc.subcore_barrier()

        pltpu.sync_copy(agg_sh.at[pl.ds(row0, ROWS_PER_SUB)],
                        out_hbm.at[core, pl.ds(row0, ROWS_PER_SUB)])

    return body(idx_i2, idx_j2, wij, h)


def _out_body(agg_ref, w1_ref, b1_ref, w2_ref, b2_ref, o_ref):
    a = agg_ref[0, :N_ATOMS, :] + agg_ref[1, :N_ATOMS, :]
    t = jax.lax.dot_general(
        a, w1_ref[...], (((1,), (1,)), ((), ())),
        preferred_element_type=jnp.float32) + b1_ref[...]
    t = _ssp(t)
    o_ref[...] = jax.lax.dot_general(
        t, w2_ref[...], (((1,), (1,)), ((), ())),
        preferred_element_type=jnp.float32) + b2_ref[...]


def _compute_out(agg, W_o1, b_o1, W_o2, b_o2):
    return pl.pallas_call(
        _out_body,
        out_shape=jax.ShapeDtypeStruct((N_ATOMS, D), jnp.float32),
    )(agg, W_o1, b_o1.reshape(1, D), W_o2, b_o2.reshape(1, D))


def kernel(x, f_ij, idx_i, idx_j, rcut_ij,
           W_in, b_in, W_f1, b_f1, W_f2, b_f2,
           W_o1, b_o1, W_o2, b_o2):
    rcut_p = rcut_ij.reshape(E_PAD, 1)
    idx_i2 = idx_i.astype(jnp.int32).reshape(E_PAD // CHUNK, CHUNK)
    idx_j2 = idx_j.astype(jnp.int32).reshape(E_PAD // CHUNK, CHUNK)

    h = _compute_h(x, W_in, b_in)
    wij = _compute_wij(f_ij, rcut_p, W_f1, b_f1, W_f2, b_f2)
    agg = _sc_aggregate(idx_i2, idx_j2, wij, h)
    return _compute_out(agg, W_o1, b_o1, W_o2, b_o2)


# speedup vs baseline: 1.1107x; 1.1107x over previous
import dataclasses
import functools

import jax
import jax.numpy as jnp
from jax.experimental import pallas as pl
from jax.experimental.pallas import tpu as pltpu
from jax.experimental.pallas import tpu_sc as plsc

N_ATOMS = 10000
N_EDGES = 320000
D = 128
N_RBF = 20

NUM_SC = 2
NUM_SUBCORES = 16
NUM_WORKERS = NUM_SC * NUM_SUBCORES
CHUNK = 80
CHUNKS_PER_WORKER = 125
E_PAD = NUM_WORKERS * CHUNKS_PER_WORKER * CHUNK
A_PAD = 10112
ROWS_PER_SUB = A_PAD // NUM_SUBCORES

_LN2 = 0.6931471805599453


def _ssp(v):
    return jnp.maximum(v, 0.0) + jnp.log1p(jnp.exp(-jnp.abs(v))) - _LN2


def _h_body(x_ref, w_ref, b_ref, o_ref):
    o_ref[...] = jax.lax.dot_general(
        x_ref[...], w_ref[...], (((1,), (1,)), ((), ())),
        preferred_element_type=jnp.float32) + b_ref[...]


def _compute_h(x, W_in, b_in):
    return pl.pallas_call(
        _h_body,
        out_shape=jax.ShapeDtypeStruct((N_ATOMS, D), jnp.float32),
    )(x, W_in, b_in.reshape(1, D))


_BE = 2000


def _filter_body(fij_ref, rcut_ref, w1_ref, b1_ref, w2_ref, b2_ref, o_ref):
    t = jax.lax.dot_general(
        fij_ref[...].astype(jnp.bfloat16),
        w1_ref[...].astype(jnp.bfloat16), (((1,), (1,)), ((), ())),
        preferred_element_type=jnp.float32) + b1_ref[...]
    t = _ssp(t)
    w = jax.lax.dot_general(
        t.astype(jnp.bfloat16),
        w2_ref[...].astype(jnp.bfloat16), (((1,), (1,)), ((), ())),
        preferred_element_type=jnp.float32) + b2_ref[...]
    o_ref[...] = w * rcut_ref[...]


def _compute_wij(f_ij_p, rcut_p, W_f1, b_f1, W_f2, b_f2):
    grid = (E_PAD // _BE,)
    return pl.pallas_call(
        _filter_body,
        grid=grid,
        in_specs=[
            pl.BlockSpec((_BE, N_RBF), lambda i: (i, 0)),
            pl.BlockSpec((_BE, 1), lambda i: (i, 0)),
            pl.BlockSpec((D, N_RBF), lambda i: (0, 0)),
            pl.BlockSpec((1, D), lambda i: (0, 0)),
            pl.BlockSpec((D, D), lambda i: (0, 0)),
            pl.BlockSpec((1, D), lambda i: (0, 0)),
        ],
        out_specs=pl.BlockSpec((_BE, D), lambda i: (i, 0)),
        out_shape=jax.ShapeDtypeStruct((E_PAD, D), jnp.float32),
    )(f_ij_p, rcut_p, W_f1, b_f1.reshape(1, D), W_f2, b_f2.reshape(1, D))


def _sc_compiler_params():
    cp = pltpu.CompilerParams()
    if "needs_layout_passes" in pltpu.CompilerParams.__dataclass_fields__:
        cp = dataclasses.replace(cp, needs_layout_passes=False)
    return cp


def _sc_aggregate(idx_i2, idx_j2, wij, h):
    mesh = plsc.VectorSubcoreMesh(
        core_axis_name="core", subcore_axis_name="subcore",
        num_cores=NUM_SC, num_subcores=NUM_SUBCORES)

    @functools.partial(
        pl.kernel,
        out_type=jax.ShapeDtypeStruct((NUM_SC, A_PAD, D), jnp.float32),
        mesh=mesh,
        scratch_types=[
            pltpu.VMEM_SHARED((A_PAD, D), jnp.float32),
            [pltpu.VMEM((CHUNK, D), jnp.float32)] * 2,
            [pltpu.VMEM((CHUNK, D), jnp.float32)] * 2,
            [pltpu.VMEM((CHUNK,), jnp.int32)] * 4,
            [pltpu.VMEM((CHUNK,), jnp.int32)] * 4,
            [pltpu.SemaphoreType.DMA] * 2,
            [pltpu.SemaphoreType.DMA] * 2,
            [pltpu.SemaphoreType.DMA] * 2,
            [pltpu.SemaphoreType.DMA] * 4,
        ],
        compiler_params=_sc_compiler_params(),
    )
    def body(idxi_hbm, idxj_hbm, wij_hbm, h_hbm, out_hbm,
             agg_sh, gbufs, wbufs, ib, jb, sg, sw, ss, si):
        core = jax.lax.axis_index("core")
        sub = jax.lax.axis_index("subcore")
        wid = core * NUM_SUBCORES + sub
        base = wid * CHUNKS_PER_WORKER

        zero16 = jnp.zeros((16,), jnp.float32)
        g0 = gbufs[0]

        @plsc.parallel_loop(0, CHUNK, unroll=2)
        def _(r):
            for l in range(0, D, 16):
                g0[r, pl.ds(l, 16)] = zero16

        row0 = sub * ROWS_PER_SUB

        @pl.loop(0, ROWS_PER_SUB // CHUNK)
        def _(k):
            pltpu.sync_copy(g0, agg_sh.at[pl.ds(row0 + k * CHUNK, CHUNK)])

        _rem = ROWS_PER_SUB % CHUNK
        if _rem:
            pltpu.sync_copy(
                g0.at[pl.ds(0, _rem)],
                agg_sh.at[pl.ds(row0 + (ROWS_PER_SUB // CHUNK) * CHUNK, _rem)])

        plsc.subcore_barrier()

        def issue_idx(c, m4):
            pltpu.async_copy(idxi_hbm.at[base + c], ib[m4], si[m4])
            pltpu.async_copy(idxj_hbm.at[base + c], jb[m4], si[m4])

        def wait_idx(m4):
            pltpu.make_async_copy(idxi_hbm.at[0], ib[m4], si[m4]).wait()
            pltpu.make_async_copy(idxj_hbm.at[0], jb[m4], si[m4]).wait()

        def issue_in(c, m2, m4):
            pltpu.async_copy(h_hbm.at[jb[m4]], gbufs[m2], sg[m2])
            pltpu.async_copy(
                wij_hbm.at[pl.ds((base + c) * CHUNK, CHUNK)], wbufs[m2],
                sw[m2])

        def wait_in(m2):
            pltpu.make_async_copy(
                h_hbm.at[pl.ds(0, CHUNK)], gbufs[m2], sg[m2]).wait()
            pltpu.make_async_copy(
                wij_hbm.at[pl.ds(0, CHUNK)], wbufs[m2], sw[m2]).wait()

        def mul(m2):
            g, w = gbufs[m2], wbufs[m2]

            @plsc.parallel_loop(0, CHUNK, unroll=2)
            def _(r):
                for l in range(0, D, 16):
                    g[r, pl.ds(l, 16)] = g[r, pl.ds(l, 16)] * w[r, pl.ds(l, 16)]

        def issue_scatter(m2, m4):
            pltpu.async_copy(gbufs[m2], agg_sh.at[ib[m4]], ss[m2], add=True)

        def wait_scatter(m2):
            pltpu.make_async_copy(
                h_hbm.at[pl.ds(0, CHUNK)], gbufs[m2], ss[m2]).wait()

        NC = CHUNKS_PER_WORKER

        def slot(c, m2, m4, sc_wait=True, idx_wait=True,
                 prefetch_in=True, prefetch_idx=True):
            wait_in(m2)
            if sc_wait:
                wait_scatter(1 - m2)
            if prefetch_in:
                if idx_wait:
                    wait_idx((m4 + 1) % 4)
                issue_in(c + 1, 1 - m2, (m4 + 1) % 4)
            mul(m2)
            issue_scatter(m2, m4)
            if prefetch_idx:
                issue_idx(c + 2, (m4 + 2) % 4)

        pltpu.sync_copy(idxi_hbm.at[base], ib[0])
        pltpu.sync_copy(idxj_hbm.at[base], jb[0])
        pltpu.sync_copy(idxi_hbm.at[base + 1], ib[1])
        pltpu.sync_copy(idxj_hbm.at[base + 1], jb[1])
        issue_in(0, 0, 0)
        slot(0, 0, 0, sc_wait=False, idx_wait=False)
        slot(1, 1, 1)

        @pl.loop(0, (NC - 5) // 4)
        def _(k):
            c = 4 * k + 2
            slot(c, 0, 2)
            slot(c + 1, 1, 3)
            slot(c + 2, 0, 0)
            slot(c + 3, 1, 1)

        slot(NC - 3, 0, 2)
        slot(NC - 2, 1, 3, prefetch_idx=False)
        slot(NC - 1, 0, 0, prefetch_in=False, prefetch_idx=False)
        wait_scatter(0)

        plsc.subcore_barrier()

        pltpu.sync_copy(agg_sh.at[pl.ds(row0, ROWS_PER_SUB)],
                        out_hbm.at[core, pl.ds(row0, ROWS_PER_SUB)])

    return body(idx_i2, idx_j2, wij, h)


def _out_body(agg_ref, w1_ref, b1_ref, w2_ref, b2_ref, o_ref):
    a = agg_ref[0, :N_ATOMS, :] + agg_ref[1, :N_ATOMS, :]
    t = jax.lax.dot_general(
        a, w1_ref[...], (((1,), (1,)), ((), ())),
        preferred_element_type=jnp.float32) + b1_ref[...]
    t = _ssp(t)
    o_ref[...] = jax.lax.dot_general(
        t, w2_ref[...], (((1,), (1,)), ((), ())),
        preferred_element_type=jnp.float32) + b2_ref[...]


def _compute_out(agg, W_o1, b_o1, W_o2, b_o2):
    return pl.pallas_call(
        _out_body,
        out_shape=jax.ShapeDtypeStruct((N_ATOMS, D), jnp.float32),
    )(agg, W_o1, b_o1.reshape(1, D), W_o2, b_o2.reshape(1, D))


def kernel(x, f_ij, idx_i, idx_j, rcut_ij,
           W_in, b_in, W_f1, b_f1, W_f2, b_f2,
           W_o1, b_o1, W_o2, b_o2):
    rcut_p = rcut_ij.reshape(E_PAD, 1)
    idx_i2 = idx_i.astype(jnp.int32).reshape(E_PAD // CHUNK, CHUNK)
    idx_j2 = idx_j.astype(jnp.int32).reshape(E_PAD // CHUNK, CHUNK)

    h = _compute_h(x, W_in, b_in)
    wij = _compute_wij(f_ij, rcut_p, W_f1, b_f1, W_f2, b_f2)
    agg = _sc_aggregate(idx_i2, idx_j2, wij, h)
    return _compute_out(agg, W_o1, b_o1, W_o2, b_o2)


# speedup vs baseline: 1.2195x; 1.0979x over previous
import dataclasses
import functools

import jax
import jax.numpy as jnp
from jax.experimental import pallas as pl
from jax.experimental.pallas import tpu as pltpu
from jax.experimental.pallas import tpu_sc as plsc

N_ATOMS = 10000
N_EDGES = 320000
D = 128
N_RBF = 20

NUM_SC = 2
NUM_SUBCORES = 16
NUM_WORKERS = NUM_SC * NUM_SUBCORES
CHUNK = 80
CHUNKS_PER_WORKER = 125
E_PAD = NUM_WORKERS * CHUNKS_PER_WORKER * CHUNK
A_PAD = 10112
ROWS_PER_SUB = A_PAD // NUM_SUBCORES

_LN2 = 0.6931471805599453


def _ssp(v):
    return jnp.maximum(v, 0.0) + jnp.log1p(jnp.exp(-jnp.abs(v))) - _LN2


def _h_body(x_ref, w_ref, b_ref, o_ref):
    o_ref[...] = jax.lax.dot_general(
        x_ref[...], w_ref[...], (((1,), (1,)), ((), ())),
        preferred_element_type=jnp.float32) + b_ref[...]


def _compute_h(x, W_in, b_in):
    return pl.pallas_call(
        _h_body,
        out_shape=jax.ShapeDtypeStruct((N_ATOMS, D), jnp.float32),
    )(x, W_in, b_in.reshape(1, D))


_BE = 4000


def _filter_body(fij_ref, rcut_ref, w1_ref, b1_ref, w2_ref, b2_ref, o_ref):
    t = jax.lax.dot_general(
        fij_ref[...].astype(jnp.bfloat16),
        w1_ref[...].astype(jnp.bfloat16), (((1,), (1,)), ((), ())),
        preferred_element_type=jnp.float32) + b1_ref[...]
    t = _ssp(t)
    w = jax.lax.dot_general(
        t.astype(jnp.bfloat16),
        w2_ref[...].astype(jnp.bfloat16), (((1,), (1,)), ((), ())),
        preferred_element_type=jnp.float32) + b2_ref[...]
    o_ref[...] = w * rcut_ref[...]


def _compute_wij(f_ij_p, rcut_p, W_f1, b_f1, W_f2, b_f2):
    grid = (E_PAD // _BE,)
    return pl.pallas_call(
        _filter_body,
        grid=grid,
        in_specs=[
            pl.BlockSpec((_BE, N_RBF), lambda i: (i, 0)),
            pl.BlockSpec((_BE, 1), lambda i: (i, 0)),
            pl.BlockSpec((D, N_RBF), lambda i: (0, 0)),
            pl.BlockSpec((1, D), lambda i: (0, 0)),
            pl.BlockSpec((D, D), lambda i: (0, 0)),
            pl.BlockSpec((1, D), lambda i: (0, 0)),
        ],
        out_specs=pl.BlockSpec((_BE, D), lambda i: (i, 0)),
        out_shape=jax.ShapeDtypeStruct((E_PAD, D), jnp.float32),
    )(f_ij_p, rcut_p, W_f1, b_f1.reshape(1, D), W_f2, b_f2.reshape(1, D))


def _sc_compiler_params():
    cp = pltpu.CompilerParams()
    if "needs_layout_passes" in pltpu.CompilerParams.__dataclass_fields__:
        cp = dataclasses.replace(cp, needs_layout_passes=False)
    return cp


def _sc_aggregate(idx_i2, idx_j2, wij, h):
    mesh = plsc.VectorSubcoreMesh(
        core_axis_name="core", subcore_axis_name="subcore",
        num_cores=NUM_SC, num_subcores=NUM_SUBCORES)

    @functools.partial(
        pl.kernel,
        out_type=jax.ShapeDtypeStruct((NUM_SC, A_PAD, D), jnp.float32),
        mesh=mesh,
        scratch_types=[
            pltpu.VMEM_SHARED((A_PAD, D), jnp.float32),
            [pltpu.VMEM((CHUNK, D), jnp.float32)] * 2,
            [pltpu.VMEM((CHUNK, D), jnp.float32)] * 2,
            [pltpu.VMEM((CHUNK,), jnp.int32)] * 4,
            [pltpu.VMEM((CHUNK,), jnp.int32)] * 4,
            [pltpu.SemaphoreType.DMA] * 2,
            [pltpu.SemaphoreType.DMA] * 2,
            [pltpu.SemaphoreType.DMA] * 2,
            [pltpu.SemaphoreType.DMA] * 4,
        ],
        compiler_params=_sc_compiler_params(),
    )
    def body(idxi_hbm, idxj_hbm, wij_hbm, h_hbm, out_hbm,
             agg_sh, gbufs, wbufs, ib, jb, sg, sw, ss, si):
        core = jax.lax.axis_index("core")
        sub = jax.lax.axis_index("subcore")
        wid = core * NUM_SUBCORES + sub
        base = wid * CHUNKS_PER_WORKER

        zero16 = jnp.zeros((16,), jnp.float32)
        g0 = gbufs[0]

        @plsc.parallel_loop(0, CHUNK, unroll=2)
        def _(r):
            for l in range(0, D, 16):
                g0[r, pl.ds(l, 16)] = zero16

        row0 = sub * ROWS_PER_SUB

        @pl.loop(0, ROWS_PER_SUB // CHUNK)
        def _(k):
            pltpu.sync_copy(g0, agg_sh.at[pl.ds(row0 + k * CHUNK, CHUNK)])

        _rem = ROWS_PER_SUB % CHUNK
        if _rem:
            pltpu.sync_copy(
                g0.at[pl.ds(0, _rem)],
                agg_sh.at[pl.ds(row0 + (ROWS_PER_SUB // CHUNK) * CHUNK, _rem)])

        plsc.subcore_barrier()

        def issue_idx(c, m4):
            off = (base + c) * CHUNK
            pltpu.async_copy(idxi_hbm.at[pl.ds(off, CHUNK)], ib[m4], si[m4])
            pltpu.async_copy(idxj_hbm.at[pl.ds(off, CHUNK)], jb[m4], si[m4])

        def wait_idx(m4):
            pltpu.make_async_copy(
                idxi_hbm.at[pl.ds(0, CHUNK)], ib[m4], si[m4]).wait()
            pltpu.make_async_copy(
                idxj_hbm.at[pl.ds(0, CHUNK)], jb[m4], si[m4]).wait()

        def issue_in(c, m2, m4):
            pltpu.async_copy(h_hbm.at[jb[m4]], gbufs[m2], sg[m2])
            pltpu.async_copy(
                wij_hbm.at[pl.ds((base + c) * CHUNK, CHUNK)], wbufs[m2],
                sw[m2])

        def wait_in(m2):
            pltpu.make_async_copy(
                h_hbm.at[pl.ds(0, CHUNK)], gbufs[m2], sg[m2]).wait()
            pltpu.make_async_copy(
                wij_hbm.at[pl.ds(0, CHUNK)], wbufs[m2], sw[m2]).wait()

        def mul(m2):
            g, w = gbufs[m2], wbufs[m2]

            @plsc.parallel_loop(0, CHUNK, unroll=2)
            def _(r):
                for l in range(0, D, 16):
                    g[r, pl.ds(l, 16)] = g[r, pl.ds(l, 16)] * w[r, pl.ds(l, 16)]

        def issue_scatter(m2, m4):
            pltpu.async_copy(gbufs[m2], agg_sh.at[ib[m4]], ss[m2], add=True)

        def wait_scatter(m2):
            pltpu.make_async_copy(
                h_hbm.at[pl.ds(0, CHUNK)], gbufs[m2], ss[m2]).wait()

        NC = CHUNKS_PER_WORKER

        def slot(c, m2, m4, sc_wait=True, idx_wait=True,
                 prefetch_in=True, prefetch_idx=True):
            wait_in(m2)
            if sc_wait:
                wait_scatter(1 - m2)
            if prefetch_in:
                if idx_wait:
                    wait_idx((m4 + 1) % 4)
                issue_in(c + 1, 1 - m2, (m4 + 1) % 4)
            mul(m2)
            issue_scatter(m2, m4)
            if prefetch_idx:
                issue_idx(c + 2, (m4 + 2) % 4)

        pltpu.sync_copy(idxi_hbm.at[pl.ds(base * CHUNK, CHUNK)], ib[0])
        pltpu.sync_copy(idxj_hbm.at[pl.ds(base * CHUNK, CHUNK)], jb[0])
        pltpu.sync_copy(idxi_hbm.at[pl.ds((base + 1) * CHUNK, CHUNK)], ib[1])
        pltpu.sync_copy(idxj_hbm.at[pl.ds((base + 1) * CHUNK, CHUNK)], jb[1])
        issue_in(0, 0, 0)
        slot(0, 0, 0, sc_wait=False, idx_wait=False)
        slot(1, 1, 1)

        @pl.loop(0, (NC - 5) // 4)
        def _(k):
            c = 4 * k + 2
            slot(c, 0, 2)
            slot(c + 1, 1, 3)
            slot(c + 2, 0, 0)
            slot(c + 3, 1, 1)

        slot(NC - 3, 0, 2)
        slot(NC - 2, 1, 3, prefetch_idx=False)
        slot(NC - 1, 0, 0, prefetch_in=False, prefetch_idx=False)
        wait_scatter(0)

        plsc.subcore_barrier()

        pltpu.sync_copy(agg_sh.at[pl.ds(row0, ROWS_PER_SUB)],
                        out_hbm.at[core, pl.ds(row0, ROWS_PER_SUB)])

    return body(idx_i2, idx_j2, wij, h)


def _out_body(agg_ref, w1_ref, b1_ref, w2_ref, b2_ref, o_ref):
    a = agg_ref[0, :N_ATOMS, :] + agg_ref[1, :N_ATOMS, :]
    t = jax.lax.dot_general(
        a, w1_ref[...], (((1,), (1,)), ((), ())),
        preferred_element_type=jnp.float32) + b1_ref[...]
    t = _ssp(t)
    o_ref[...] = jax.lax.dot_general(
        t, w2_ref[...], (((1,), (1,)), ((), ())),
        preferred_element_type=jnp.float32) + b2_ref[...]


def _compute_out(agg, W_o1, b_o1, W_o2, b_o2):
    return pl.pallas_call(
        _out_body,
        out_shape=jax.ShapeDtypeStruct((N_ATOMS, D), jnp.float32),
    )(agg, W_o1, b_o1.reshape(1, D), W_o2, b_o2.reshape(1, D))


def kernel(x, f_ij, idx_i, idx_j, rcut_ij,
           W_in, b_in, W_f1, b_f1, W_f2, b_f2,
           W_o1, b_o1, W_o2, b_o2):
    rcut_p = rcut_ij.reshape(E_PAD, 1)
    idx_i2 = idx_i.astype(jnp.int32)
    idx_j2 = idx_j.astype(jnp.int32)

    h = _compute_h(x, W_in, b_in)
    wij = _compute_wij(f_ij, rcut_p, W_f1, b_f1, W_f2, b_f2)
    agg = _sc_aggregate(idx_i2, idx_j2, wij, h)
    return _compute_out(agg, W_o1, b_o1, W_o2, b_o2)
